# baseline (speedup 1.0000x reference)
import functools

import jax
import jax.numpy as jnp
from jax import lax
from jax.experimental import pallas as pl
from jax.experimental.pallas import tpu as pltpu
from jax.experimental.pallas import tpu_sc as plsc

B = 8
N = 4096
S = 1024
K = 32
RADIUS = 0.2


_G = 8
_L = N // _G


def _grp(p, red, bmat):
    f32 = p.dtype == jnp.float32
    q = p.reshape(B, _G)
    t = red(q, axis=1, keepdims=True)
    tf = t if f32 else t.astype(jnp.float32)
    full = lax.dot_general(bmat, tf, (((1,), (0,)), ((), ())),
                           preferred_element_type=jnp.float32)
    return t, (full if f32 else full.astype(p.dtype))


def _fps_body(x_ref, y_ref, z_ref, idx_ref, cx_ref, cy_ref, cz_ref, dist_ref):
    BG = B * _G
    dist_ref[...] = jnp.full((BG, _L), 1e10, jnp.float32)
    gi = jax.lax.broadcasted_iota(jnp.int32, (BG, _L), 0) & (_G - 1)
    iota = gi * _L + jax.lax.broadcasted_iota(jnp.int32, (BG, _L), 1)
    lane = jax.lax.broadcasted_iota(jnp.int32, (B, 128), 1)
    bmat = (jax.lax.broadcasted_iota(jnp.int32, (BG, B), 0) // _G
            == jax.lax.broadcasted_iota(jnp.int32, (BG, B), 1)
            ).astype(jnp.float32)
    x = x_ref[...]
    y = y_ref[...]
    z = z_ref[...]

    def outer(j, far0):
        z32 = jnp.zeros((B, 128), jnp.int32)
        zf = jnp.zeros((B, 128), jnp.float32)

        def inner(t, st):
            far64, far8, sidx, scx, scy, scz = st
            ms = jnp.where(iota == far64, 1.0, 0.0)
            cx8, cx = _grp(jnp.sum(x * ms, axis=1, keepdims=True), jnp.sum, bmat)
            cy8, cy = _grp(jnp.sum(y * ms, axis=1, keepdims=True), jnp.sum, bmat)
            cz8, cz = _grp(jnp.sum(z * ms, axis=1, keepdims=True), jnp.sum, bmat)
            dx = x - cx
            dy = y - cy
            dz = z - cz
            d = dx * dx + dy * dy + dz * dz
            dist = jnp.minimum(dist_ref[...], d)
            dist_ref[...] = dist
            _, mx = _grp(jnp.max(dist, axis=1, keepdims=True), jnp.max, bmat)
            nfp = jnp.min(jnp.where(dist == mx, iota, N), axis=1,
                          keepdims=True)
            nf8, nf64 = _grp(nfp, jnp.min, bmat)
            oh = lane == t
            sidx = jnp.where(oh, far8, sidx)
            scx = jnp.where(oh, cx8, scx)
            scy = jnp.where(oh, cy8, scy)
            scz = jnp.where(oh, cz8, scz)
            return (nf64.astype(jnp.int32), nf8.astype(jnp.int32),
                    sidx, scx, scy, scz)

        st = jax.lax.fori_loop(
            0, 128, inner,
            (far0[0], far0[1], z32, zf, zf, zf), unroll=False)
        far64, far8, sidx, scx, scy, scz = st
        col = pl.multiple_of(j * 128, 128)
        idx_ref[:, pl.ds(col, 128)] = sidx
        cx_ref[:, pl.ds(col, 128)] = scx
        cy_ref[:, pl.ds(col, 128)] = scy
        cz_ref[:, pl.ds(col, 128)] = scz
        return (far64, far8)

    jax.lax.fori_loop(0, S // 128, outer,
                      (jnp.zeros((B * _G, 1), jnp.int32),
                       jnp.zeros((B, 1), jnp.int32)),
                      unroll=False)


@functools.partial(jax.jit, static_argnames=("interpret",))
def _fps(x, y, z, interpret=False):
    out_shapes = (
        jax.ShapeDtypeStruct((B, S), jnp.int32),
        jax.ShapeDtypeStruct((B, S), jnp.float32),
        jax.ShapeDtypeStruct((B, S), jnp.float32),
        jax.ShapeDtypeStruct((B, S), jnp.float32),
    )
    return pl.pallas_call(
        _fps_body,
        out_shape=out_shapes,
        scratch_shapes=[pltpu.VMEM((B * _G, _L), jnp.float32)],
        interpret=interpret,
    )(x.reshape(B * _G, _L), y.reshape(B * _G, _L), z.reshape(B * _G, _L))


_NW = 32
_RPW = (B * S) // _NW
_RBLK = 16
_CB = 16
_NBLK2 = N // (16 * _CB)


def _bq_body(px_h, py_h, pz_h, cx_h, cy_h, cz_h, out_h,
             x_v, y_v, z_v, cx_v, cy_v, cz_v, blk_v):
    w = lax.axis_index("s") * 2 + lax.axis_index("c")
    b = w // (_NW // B)
    row0 = w * _RPW
    pltpu.sync_copy(px_h.at[pl.ds(b * N, N)], x_v)
    pltpu.sync_copy(py_h.at[pl.ds(b * N, N)], y_v)
    pltpu.sync_copy(pz_h.at[pl.ds(b * N, N)], z_v)
    pltpu.sync_copy(cx_h.at[pl.ds(row0, _RPW)], cx_v)
    pltpu.sync_copy(cy_h.at[pl.ds(row0, _RPW)], cy_v)
    pltpu.sync_copy(cz_h.at[pl.ds(row0, _RPW)], cz_v)
    r2 = RADIUS * RADIUS
    iota16 = lax.iota(jnp.int32, 16)
    boff = b * N

    def do_block(blk, _):
        def do_row(r, _):
            sw = blk * _RBLK + r
            rowoff = r * K
            sv = jnp.full((16,), sw, jnp.int32)
            cxs = plsc.load_gather(cx_v, [sv])
            cys = plsc.load_gather(cy_v, [sv])
            czs = plsc.load_gather(cz_v, [sv])

            def cond(st):
                cnt, blk2 = st
                return (cnt < K) & (blk2 < _NBLK2)

            def step(st):
                cnt, blk2 = st
                base0 = blk2 * (16 * _CB)
                masks = []
                for c in range(_CB):
                    off = base0 + c * 16
                    dx = x_v[pl.ds(off, 16)] - cxs
                    dy = y_v[pl.ds(off, 16)] - cys
                    dz = z_v[pl.ds(off, 16)] - czs
                    masks.append((dx * dx + dy * dy + dz * dz) <= r2)
                basev = jnp.full((16,), cnt, jnp.int32)
                bases = []
                for c in range(_CB):
                    bases.append(basev)
                    basev = basev + plsc.all_reduce_population_count(masks[c])
                for c in range(_CB):
                    m = masks[c]
                    pc = plsc.cumsum(jnp.where(m, 1, 0).astype(jnp.int32))
                    pos = pc + (bases[c] - 1)
                    wmask = m & (pos < K)
                    vals = (boff + base0 + c * 16) + iota16
                    plsc.store_scatter(blk_v, [pos + rowoff], vals, mask=wmask)
                return (jnp.max(basev), blk2 + 1)

            cnt, _ = lax.while_loop(cond, step, (jnp.int32(0), jnp.int32(0)))

            first = plsc.load_gather(blk_v, [jnp.full((16,), rowoff, jnp.int32)])
            for h in range(2):
                off = rowoff + h * 16
                cur = blk_v[pl.ds(off, 16)]
                lanes = iota16 + (h * 16)
                blk_v[pl.ds(off, 16)] = jnp.where(lanes >= cnt, first, cur)
            return _

        lax.fori_loop(0, _RBLK, do_row, 0, unroll=False)
        pltpu.sync_copy(blk_v, out_h.at[pl.ds((row0 + blk * _RBLK) * K,
                                              _RBLK * K)])
        return _

    lax.fori_loop(0, _RPW // _RBLK, do_block, 0, unroll=False)


@jax.jit
def _ball_query_sc(px, py, pz, cx, cy, cz):
    mesh = plsc.VectorSubcoreMesh(core_axis_name="c", subcore_axis_name="s")
    f = pl.kernel(
        _bq_body,
        mesh=mesh,
        compiler_params=pltpu.CompilerParams(needs_layout_passes=False),
        out_type=jax.ShapeDtypeStruct((B * S * K,), jnp.int32),
        scratch_types=[
            pltpu.VMEM((N,), jnp.float32),
            pltpu.VMEM((N,), jnp.float32),
            pltpu.VMEM((N,), jnp.float32),
            pltpu.VMEM((_RPW,), jnp.float32),
            pltpu.VMEM((_RPW,), jnp.float32),
            pltpu.VMEM((_RPW,), jnp.float32),
            pltpu.VMEM((_RBLK * K,), jnp.int32),
        ],
    )
    return f(px, py, pz, cx, cy, cz)


_R = B * S * K
_EPS = 1e-5


def _t1_body(xt_ref, f_ref, wx_ref, wf_ref, out_ref):
    yx = lax.dot_general(xt_ref[0], wx_ref[...], (((0,), (1,)), ((), ())),
                         preferred_element_type=jnp.float32)
    yf = lax.dot_general(f_ref[0], wf_ref[...], (((0,), (1,)), ((), ())),
                         preferred_element_type=jnp.float32)
    out_ref[0] = jnp.concatenate(
        [yx + yf, jnp.zeros((512, 64), jnp.float32)], axis=1)


@jax.jit
def _t1_table(xt, features, wx, wf):
    return pl.pallas_call(
        _t1_body,
        grid=(B, N // 512),
        in_specs=[
            pl.BlockSpec((1, 3, 512), lambda b, j: (b, 0, j)),
            pl.BlockSpec((1, 64, 512), lambda b, j: (b, 0, j)),
            pl.BlockSpec((64, 3), lambda b, j: (0, 0)),
            pl.BlockSpec((64, 64), lambda b, j: (0, 0)),
        ],
        out_specs=pl.BlockSpec((1, 512, 128), lambda b, j: (b, j, 0)),
        out_shape=jax.ShapeDtypeStruct((B, N, 128), jnp.float32),
    )(xt, features, wx, wf)


def _q_body(c_ref, wx_ref, out_ref):
    out_ref[0] = lax.dot_general(c_ref[0], wx_ref[...], (((0,), (1,)), ((), ())),
                                 preferred_element_type=jnp.float32)


@jax.jit
def _q_table(cstack, wx):
    return pl.pallas_call(
        _q_body,
        grid=(B, S // 512),
        in_specs=[
            pl.BlockSpec((1, 3, 512), lambda b, j: (b, 0, j)),
            pl.BlockSpec((64, 3), lambda b, j: (0, 0)),
        ],
        out_specs=pl.BlockSpec((1, 512, 64), lambda b, j: (b, j, 0)),
        out_shape=jax.ShapeDtypeStruct((B, S, 64), jnp.float32),
    )(cstack, wx)


_GCH = _R // _NW
_GBLK = 128


def _gather_body(t1_h, gidx_h, out_h, idx_v, buf_v, sem):
    w = lax.axis_index("s") * 2 + lax.axis_index("c")
    base = w * _GCH

    def do_chunk(t, _):
        pltpu.sync_copy(gidx_h.at[pl.ds(base + t * _GBLK, _GBLK)], idx_v)
        pltpu.async_copy(t1_h.at[idx_v], buf_v, sem).wait()
        pltpu.sync_copy(buf_v, out_h.at[pl.ds(base + t * _GBLK, _GBLK)])
        return _

    lax.fori_loop(0, _GCH // _GBLK, do_chunk, 0, unroll=False)


@jax.jit
def _gather_sc(t1_flat, gidx):
    mesh = plsc.VectorSubcoreMesh(core_axis_name="c", subcore_axis_name="s")
    f = pl.kernel(
        _gather_body,
        mesh=mesh,
        compiler_params=pltpu.CompilerParams(needs_layout_passes=False),
        out_type=jax.ShapeDtypeStruct((_R, 128), jnp.float32),
        scratch_types=[
            pltpu.VMEM((_GBLK,), jnp.int32),
            pltpu.VMEM((_GBLK, 128), jnp.float32),
            pltpu.SemaphoreType.DMA,
        ],
    )
    return f(t1_flat, gidx)


_TR = 2048
_NT = _R // _TR


def _bn_coeffs(st_ref, g_ref, b_ref):
    s = st_ref[0, :]
    sq = st_ref[1, :]
    m = s / _R
    var = sq / _R - m * m
    inv = g_ref[0, :] / jnp.sqrt(var + _EPS)
    return inv, b_ref[0, :] - m * inv


def _x1_tile(g_ref, q_ref, st1_ref, g1_ref, b1_ref):
    sc1, sh1 = _bn_coeffs(st1_ref, g1_ref, b1_ref)
    y1 = g_ref[:, 0:64].reshape(_TR // K, K, 64) - q_ref[...][:, None, :]
    x1 = jnp.maximum(y1 * sc1[None, None, :] + sh1[None, None, :], 0.0)
    return x1.reshape(_TR, 64)


def _stats1_body(g_ref, q_ref, out_ref):
    y = g_ref[:, 0:64].reshape(_TR // K, K, 64) - q_ref[...][:, None, :]
    s = jnp.sum(y, axis=(0, 1))
    sq = jnp.sum(y * y, axis=(0, 1))
    st = jnp.stack([s, sq])

    @pl.when(pl.program_id(0) == 0)
    def _():
        out_ref[...] = st

    @pl.when(pl.program_id(0) != 0)
    def _():
        out_ref[...] += st


@jax.jit
def _stats1(G, Q):
    return pl.pallas_call(
        _stats1_body,
        grid=(_NT,),
        in_specs=[
            pl.BlockSpec((_TR, 128), lambda i: (i, 0)),
            pl.BlockSpec((_TR // K, 64), lambda i: (i, 0)),
        ],
        out_specs=pl.BlockSpec((2, 64), lambda i: (0, 0)),
        out_shape=jax.ShapeDtypeStruct((2, 64), jnp.float32),
    )(G, Q)


def _stats2_body(g_ref, q_ref, st1_ref, g1_ref, b1_ref, out_ref):
    x1 = _x1_tile(g_ref, q_ref, st1_ref, g1_ref, b1_ref)
    gram = lax.dot_general(x1, x1, (((0,), (0,)), ((), ())),
                           preferred_element_type=jnp.float32)
    cs = jnp.sum(x1, axis=0)
    st = jnp.concatenate([gram, cs[None, :]], axis=0)

    @pl.when(pl.program_id(0) == 0)
    def _():
        out_ref[...] = st

    @pl.when(pl.program_id(0) != 0)
    def _():
        out_ref[...] += st


@jax.jit
def _stats2(G, Q, st1, g1, b1):
    return pl.pallas_call(
        _stats2_body,
        grid=(_NT,),
        in_specs=[
            pl.BlockSpec((_TR, 128), lambda i: (i, 0)),
            pl.BlockSpec((_TR // K, 64), lambda i: (i, 0)),
            pl.BlockSpec((2, 64), lambda i: (0, 0)),
            pl.BlockSpec((1, 64), lambda i: (0, 0)),
            pl.BlockSpec((1, 64), lambda i: (0, 0)),
        ],
        out_specs=pl.BlockSpec((65, 64), lambda i: (0, 0)),
        out_shape=jax.ShapeDtypeStruct((65, 64), jnp.float32),
    )(G, Q, st1, g1, b1)


def _mlp_body(g_ref, q_ref, st1_ref, gs_ref, g1_ref, b1_ref, g2_ref, b2_ref,
              w2_ref, w3_ref, m3_ref, st3_ref):
    x1 = _x1_tile(g_ref, q_ref, st1_ref, g1_ref, b1_ref)

    w2 = w2_ref[...]
    cs = gs_ref[64, :]
    gram = gs_ref[0:64, :]
    m2 = lax.dot_general(w2, cs.reshape(64, 1), (((1,), (0,)), ((), ())),
                         preferred_element_type=jnp.float32)[:, 0] / _R
    t = lax.dot_general(w2, gram, (((1,), (0,)), ((), ())),
                        preferred_element_type=jnp.float32)
    e2 = jnp.sum(t * w2, axis=1) / _R
    var2 = e2 - m2 * m2
    sc2 = g2_ref[0, :] / jnp.sqrt(var2 + _EPS)
    sh2 = b2_ref[0, :] - m2 * sc2

    y2 = lax.dot_general(x1, w2, (((1,), (1,)), ((), ())),
                         preferred_element_type=jnp.float32)
    x2 = jnp.maximum(y2 * sc2[None, :] + sh2[None, :], 0.0)
    y3 = lax.dot_general(x2, w3_ref[...], (((1,), (1,)), ((), ())),
                         preferred_element_type=jnp.float32)
    s3 = jnp.sum(y3, axis=0)
    q3 = jnp.sum(y3 * y3, axis=0)
    st = jnp.stack([s3, q3])
    m3_ref[...] = jnp.max(y3.reshape(_TR // K, K, 128), axis=1)

    @pl.when(pl.program_id(0) == 0)
    def _():
        st3_ref[...] = st

    @pl.when(pl.program_id(0) != 0)
    def _():
        st3_ref[...] += st


@jax.jit
def _mlp(G, Q, st1, gs, g1, b1, g2, b2, W2, W3):
    return pl.pallas_call(
        _mlp_body,
        grid=(_NT,),
        in_specs=[
            pl.BlockSpec((_TR, 128), lambda i: (i, 0)),
            pl.BlockSpec((_TR // K, 64), lambda i: (i, 0)),
            pl.BlockSpec((2, 64), lambda i: (0, 0)),
            pl.BlockSpec((65, 64), lambda i: (0, 0)),
            pl.BlockSpec((1, 64), lambda i: (0, 0)),
            pl.BlockSpec((1, 64), lambda i: (0, 0)),
            pl.BlockSpec((1, 64), lambda i: (0, 0)),
            pl.BlockSpec((1, 64), lambda i: (0, 0)),
            pl.BlockSpec((64, 64), lambda i: (0, 0)),
            pl.BlockSpec((128, 64), lambda i: (0, 0)),
        ],
        out_specs=[
            pl.BlockSpec((_TR // K, 128), lambda i: (i, 0)),
            pl.BlockSpec((2, 128), lambda i: (0, 0)),
        ],
        out_shape=[
            jax.ShapeDtypeStruct((B * S, 128), jnp.float32),
            jax.ShapeDtypeStruct((2, 128), jnp.float32),
        ],
    )(G, Q, st1, gs, g1, b1, g2, b2, W2, W3)


def _bn3_body(m3_ref, st3_ref, g3_ref, b3_ref, out_ref):
    sc3, sh3 = _bn_coeffs(st3_ref, g3_ref, b3_ref)
    o = jnp.maximum(m3_ref[...] * sc3[None, :] + sh3[None, :], 0.0)
    out_ref[0] = o.T


@jax.jit
def _bn3(m3, st3, g3, b3):
    return pl.pallas_call(
        _bn3_body,
        grid=(B,),
        in_specs=[
            pl.BlockSpec((S, 128), lambda i: (i, 0)),
            pl.BlockSpec((2, 128), lambda i: (0, 0)),
            pl.BlockSpec((1, 128), lambda i: (0, 0)),
            pl.BlockSpec((1, 128), lambda i: (0, 0)),
        ],
        out_specs=pl.BlockSpec((1, 128, S), lambda i: (i, 0, 0)),
        out_shape=jax.ShapeDtypeStruct((B, 128, S), jnp.float32),
    )(m3, st3, g3, b3)


def kernel(xyz, features, W1, g1, b1, W2, g2, b2, W3, g3, b3):
    xt = jnp.transpose(xyz, (0, 2, 1))
    fps_idx, cx, cy, cz = _fps(xt[:, 0], xt[:, 1], xt[:, 2])
    new_xyz = jnp.stack([cx, cy, cz], axis=-1)

    gidx = _ball_query_sc(
        xt[:, 0].reshape(-1), xt[:, 1].reshape(-1), xt[:, 2].reshape(-1),
        cx.reshape(-1), cy.reshape(-1), cz.reshape(-1))

    wx = W1[:, :3]
    wf = W1[:, 3:]
    T1 = _t1_table(xt, features, wx, wf).reshape(B * N, 128)
    Q = _q_table(jnp.stack([cx, cy, cz], axis=1), wx).reshape(B * S, 64)
    G = _gather_sc(T1, gidx)

    st1 = _stats1(G, Q)
    gs = _stats2(G, Q, st1, g1[None, :], b1[None, :])
    m3, st3 = _mlp(G, Q, st1, gs, g1[None, :], b1[None, :],
                   g2[None, :], b2[None, :], W2, W3)
    new_features = _bn3(m3, st3, g3[None, :], b3[None, :])
    return (new_xyz, new_features)


# speedup vs baseline: 2.7085x; 2.7085x over previous
import functools

import jax
import jax.numpy as jnp
from jax import lax
from jax.experimental import pallas as pl
from jax.experimental.pallas import tpu as pltpu
from jax.experimental.pallas import tpu_sc as plsc

B = 8
N = 4096
S = 1024
K = 32
RADIUS = 0.2


def _fps_body(x_ref, y_ref, z_ref, idx_ref, cx_ref, cy_ref, cz_ref, dist_ref):
    dist_ref[...] = jnp.full((B, N), 1e10, jnp.float32)
    iota = jax.lax.broadcasted_iota(jnp.int32, (B, N), 1)
    lane = jax.lax.broadcasted_iota(jnp.int32, (B, 128), 1)
    x = x_ref[...]
    y = y_ref[...]
    z = z_ref[...]

    def outer(j, far):
        z32 = jnp.zeros((B, 128), jnp.int32)
        zf = jnp.zeros((B, 128), jnp.float32)

        def inner(t, st):
            far, sidx, scx, scy, scz = st
            m = iota == far
            ms = jnp.where(m, 1.0, 0.0)
            cx = jnp.sum(x * ms, axis=1, keepdims=True)
            cy = jnp.sum(y * ms, axis=1, keepdims=True)
            cz = jnp.sum(z * ms, axis=1, keepdims=True)
            dx = x - cx
            dy = y - cy
            dz = z - cz
            d = dx * dx + dy * dy + dz * dz
            dist = jnp.minimum(dist_ref[...], d)
            dist_ref[...] = dist
            mx = jnp.max(dist, axis=1, keepdims=True)
            newfar = jnp.min(jnp.where(dist == mx, iota, N), axis=1,
                             keepdims=True).astype(jnp.int32)
            oh = lane == t
            sidx = jnp.where(oh, far, sidx)
            scx = jnp.where(oh, cx, scx)
            scy = jnp.where(oh, cy, scy)
            scz = jnp.where(oh, cz, scz)
            return (newfar, sidx, scx, scy, scz)

        far, sidx, scx, scy, scz = jax.lax.fori_loop(
            0, 128, inner, (far, z32, zf, zf, zf), unroll=False)
        col = pl.multiple_of(j * 128, 128)
        idx_ref[:, pl.ds(col, 128)] = sidx
        cx_ref[:, pl.ds(col, 128)] = scx
        cy_ref[:, pl.ds(col, 128)] = scy
        cz_ref[:, pl.ds(col, 128)] = scz
        return far

    jax.lax.fori_loop(0, S // 128, outer, jnp.zeros((B, 1), jnp.int32),
                      unroll=False)


@functools.partial(jax.jit, static_argnames=("interpret",))
def _fps(x, y, z, interpret=False):
    out_shapes = (
        jax.ShapeDtypeStruct((B, S), jnp.int32),
        jax.ShapeDtypeStruct((B, S), jnp.float32),
        jax.ShapeDtypeStruct((B, S), jnp.float32),
        jax.ShapeDtypeStruct((B, S), jnp.float32),
    )
    return pl.pallas_call(
        _fps_body,
        out_shape=out_shapes,
        scratch_shapes=[pltpu.VMEM((B, N), jnp.float32)],
        interpret=interpret,
    )(x, y, z)


_NW = 32
_RPW = (B * S) // _NW
_RBLK = 16
_CB = 16
_NBLK2 = N // (16 * _CB)


def _bq_body(px_h, py_h, pz_h, cx_h, cy_h, cz_h, out_h,
             x_v, y_v, z_v, cx_v, cy_v, cz_v, blk_v):
    w = lax.axis_index("s") * 2 + lax.axis_index("c")
    b = w // (_NW // B)
    row0 = w * _RPW
    pltpu.sync_copy(px_h.at[pl.ds(b * N, N)], x_v)
    pltpu.sync_copy(py_h.at[pl.ds(b * N, N)], y_v)
    pltpu.sync_copy(pz_h.at[pl.ds(b * N, N)], z_v)
    pltpu.sync_copy(cx_h.at[pl.ds(row0, _RPW)], cx_v)
    pltpu.sync_copy(cy_h.at[pl.ds(row0, _RPW)], cy_v)
    pltpu.sync_copy(cz_h.at[pl.ds(row0, _RPW)], cz_v)
    r2 = RADIUS * RADIUS
    iota16 = lax.iota(jnp.int32, 16)
    boff = b * N

    def do_block(blk, _):
        def do_row(r, _):
            sw = blk * _RBLK + r
            rowoff = r * K
            sv = jnp.full((16,), sw, jnp.int32)
            cxs = plsc.load_gather(cx_v, [sv])
            cys = plsc.load_gather(cy_v, [sv])
            czs = plsc.load_gather(cz_v, [sv])

            def cond(st):
                cnt, blk2 = st
                return (cnt < K) & (blk2 < _NBLK2)

            def step(st):
                cnt, blk2 = st
                base0 = blk2 * (16 * _CB)
                masks = []
                for c in range(_CB):
                    off = base0 + c * 16
                    dx = x_v[pl.ds(off, 16)] - cxs
                    dy = y_v[pl.ds(off, 16)] - cys
                    dz = z_v[pl.ds(off, 16)] - czs
                    masks.append((dx * dx + dy * dy + dz * dz) <= r2)
                basev = jnp.full((16,), cnt, jnp.int32)
                bases = []
                for c in range(_CB):
                    bases.append(basev)
                    basev = basev + plsc.all_reduce_population_count(masks[c])
                for c in range(_CB):
                    m = masks[c]
                    pc = plsc.cumsum(jnp.where(m, 1, 0).astype(jnp.int32))
                    pos = pc + (bases[c] - 1)
                    wmask = m & (pos < K)
                    vals = (boff + base0 + c * 16) + iota16
                    plsc.store_scatter(blk_v, [pos + rowoff], vals, mask=wmask)
                return (jnp.max(basev), blk2 + 1)

            cnt, _ = lax.while_loop(cond, step, (jnp.int32(0), jnp.int32(0)))

            first = plsc.load_gather(blk_v, [jnp.full((16,), rowoff, jnp.int32)])
            for h in range(2):
                off = rowoff + h * 16
                cur = blk_v[pl.ds(off, 16)]
                lanes = iota16 + (h * 16)
                blk_v[pl.ds(off, 16)] = jnp.where(lanes >= cnt, first, cur)
            return _

        lax.fori_loop(0, _RBLK, do_row, 0, unroll=False)
        pltpu.sync_copy(blk_v, out_h.at[pl.ds((row0 + blk * _RBLK) * K,
                                              _RBLK * K)])
        return _

    lax.fori_loop(0, _RPW // _RBLK, do_block, 0, unroll=False)


@jax.jit
def _ball_query_sc(px, py, pz, cx, cy, cz):
    mesh = plsc.VectorSubcoreMesh(core_axis_name="c", subcore_axis_name="s")
    f = pl.kernel(
        _bq_body,
        mesh=mesh,
        compiler_params=pltpu.CompilerParams(needs_layout_passes=False),
        out_type=jax.ShapeDtypeStruct((B * S * K,), jnp.int32),
        scratch_types=[
            pltpu.VMEM((N,), jnp.float32),
            pltpu.VMEM((N,), jnp.float32),
            pltpu.VMEM((N,), jnp.float32),
            pltpu.VMEM((_RPW,), jnp.float32),
            pltpu.VMEM((_RPW,), jnp.float32),
            pltpu.VMEM((_RPW,), jnp.float32),
            pltpu.VMEM((_RBLK * K,), jnp.int32),
        ],
    )
    return f(px, py, pz, cx, cy, cz)


_R = B * S * K
_EPS = 1e-5


def _t1_body(xt_ref, f_ref, wx_ref, wf_ref, out_ref):
    yx = lax.dot_general(xt_ref[0], wx_ref[...], (((0,), (1,)), ((), ())),
                         preferred_element_type=jnp.float32)
    yf = lax.dot_general(f_ref[0], wf_ref[...], (((0,), (1,)), ((), ())),
                         preferred_element_type=jnp.float32)
    out_ref[0] = jnp.concatenate(
        [yx + yf, jnp.zeros((512, 64), jnp.float32)], axis=1)


@jax.jit
def _t1_table(xt, features, wx, wf):
    return pl.pallas_call(
        _t1_body,
        grid=(B, N // 512),
        in_specs=[
            pl.BlockSpec((1, 3, 512), lambda b, j: (b, 0, j)),
            pl.BlockSpec((1, 64, 512), lambda b, j: (b, 0, j)),
            pl.BlockSpec((64, 3), lambda b, j: (0, 0)),
            pl.BlockSpec((64, 64), lambda b, j: (0, 0)),
        ],
        out_specs=pl.BlockSpec((1, 512, 128), lambda b, j: (b, j, 0)),
        out_shape=jax.ShapeDtypeStruct((B, N, 128), jnp.float32),
    )(xt, features, wx, wf)


def _q_body(c_ref, wx_ref, out_ref):
    out_ref[0] = lax.dot_general(c_ref[0], wx_ref[...], (((0,), (1,)), ((), ())),
                                 preferred_element_type=jnp.float32)


@jax.jit
def _q_table(cstack, wx):
    return pl.pallas_call(
        _q_body,
        grid=(B, S // 512),
        in_specs=[
            pl.BlockSpec((1, 3, 512), lambda b, j: (b, 0, j)),
            pl.BlockSpec((64, 3), lambda b, j: (0, 0)),
        ],
        out_specs=pl.BlockSpec((1, 512, 64), lambda b, j: (b, j, 0)),
        out_shape=jax.ShapeDtypeStruct((B, S, 64), jnp.float32),
    )(cstack, wx)


_GCH = _R // _NW
_GBLK = 128


def _gather_body(t1_h, gidx_h, out_h, idx_v, buf_v, semg, semw):
    w = lax.axis_index("s") * 2 + lax.axis_index("c")
    base = w * _GCH
    nch = _GCH // _GBLK
    pltpu.sync_copy(gidx_h.at[pl.ds(base, _GCH)], idx_v)

    pltpu.async_copy(t1_h.at[idx_v.at[pl.ds(0, _GBLK)]], buf_v.at[0],
                     semg).wait()

    def do_chunk(t, _):
        cur = t & 1
        pltpu.async_copy(buf_v.at[1 - cur],
                         out_h.at[pl.ds(base + (t - 1) * _GBLK, _GBLK)], semw)
        pltpu.async_copy(t1_h.at[idx_v.at[pl.ds(t * _GBLK, _GBLK)]],
                         buf_v.at[cur], semg).wait()
        pltpu.make_async_copy(
            buf_v.at[cur], out_h.at[pl.ds(base, _GBLK)], semw).wait()
        return _

    lax.fori_loop(1, nch, do_chunk, 0, unroll=False)
    pltpu.sync_copy(buf_v.at[(nch - 1) & 1],
                    out_h.at[pl.ds(base + (nch - 1) * _GBLK, _GBLK)])


@jax.jit
def _gather_sc(t1_flat, gidx):
    mesh = plsc.VectorSubcoreMesh(core_axis_name="c", subcore_axis_name="s")
    f = pl.kernel(
        _gather_body,
        mesh=mesh,
        compiler_params=pltpu.CompilerParams(needs_layout_passes=False),
        out_type=jax.ShapeDtypeStruct((_R, 128), jnp.float32),
        scratch_types=[
            pltpu.VMEM((_GCH,), jnp.int32),
            pltpu.VMEM((2, _GBLK, 128), jnp.float32),
            pltpu.SemaphoreType.DMA,
            pltpu.SemaphoreType.DMA,
        ],
    )
    return f(t1_flat, gidx)


_TR = 2048
_NT = _R // _TR


def _bn_coeffs(st_ref, g_ref, b_ref):
    s = st_ref[0, :]
    sq = st_ref[1, :]
    m = s / _R
    var = sq / _R - m * m
    inv = g_ref[0, :] / jnp.sqrt(var + _EPS)
    return inv, b_ref[0, :] - m * inv


def _x1_tile(g_ref, q_ref, st1_ref, g1_ref, b1_ref):
    sc1, sh1 = _bn_coeffs(st1_ref, g1_ref, b1_ref)
    y1 = g_ref[:, 0:64].reshape(_TR // K, K, 64) - q_ref[...][:, None, :]
    x1 = jnp.maximum(y1 * sc1[None, None, :] + sh1[None, None, :], 0.0)
    return x1.reshape(_TR, 64)


def _stats1_body(g_ref, q_ref, out_ref):
    y = g_ref[:, 0:64].reshape(_TR // K, K, 64) - q_ref[...][:, None, :]
    s = jnp.sum(y, axis=(0, 1))
    sq = jnp.sum(y * y, axis=(0, 1))
    st = jnp.stack([s, sq])

    @pl.when(pl.program_id(0) == 0)
    def _():
        out_ref[...] = st

    @pl.when(pl.program_id(0) != 0)
    def _():
        out_ref[...] += st


@jax.jit
def _stats1(G, Q):
    return pl.pallas_call(
        _stats1_body,
        grid=(_NT,),
        in_specs=[
            pl.BlockSpec((_TR, 128), lambda i: (i, 0)),
            pl.BlockSpec((_TR // K, 64), lambda i: (i, 0)),
        ],
        out_specs=pl.BlockSpec((2, 64), lambda i: (0, 0)),
        out_shape=jax.ShapeDtypeStruct((2, 64), jnp.float32),
    )(G, Q)


def _stats2_body(g_ref, q_ref, st1_ref, g1_ref, b1_ref, out_ref):
    x1 = _x1_tile(g_ref, q_ref, st1_ref, g1_ref, b1_ref)
    gram = lax.dot_general(x1, x1, (((0,), (0,)), ((), ())),
                           preferred_element_type=jnp.float32)
    cs = jnp.sum(x1, axis=0)
    st = jnp.concatenate([gram, cs[None, :]], axis=0)

    @pl.when(pl.program_id(0) == 0)
    def _():
        out_ref[...] = st

    @pl.when(pl.program_id(0) != 0)
    def _():
        out_ref[...] += st


@jax.jit
def _stats2(G, Q, st1, g1, b1):
    return pl.pallas_call(
        _stats2_body,
        grid=(_NT,),
        in_specs=[
            pl.BlockSpec((_TR, 128), lambda i: (i, 0)),
            pl.BlockSpec((_TR // K, 64), lambda i: (i, 0)),
            pl.BlockSpec((2, 64), lambda i: (0, 0)),
            pl.BlockSpec((1, 64), lambda i: (0, 0)),
            pl.BlockSpec((1, 64), lambda i: (0, 0)),
        ],
        out_specs=pl.BlockSpec((65, 64), lambda i: (0, 0)),
        out_shape=jax.ShapeDtypeStruct((65, 64), jnp.float32),
    )(G, Q, st1, g1, b1)


def _mlp_body(g_ref, q_ref, st1_ref, gs_ref, g1_ref, b1_ref, g2_ref, b2_ref,
              w2_ref, w3_ref, m3_ref, st3_ref):
    x1 = _x1_tile(g_ref, q_ref, st1_ref, g1_ref, b1_ref)

    w2 = w2_ref[...]
    cs = gs_ref[64, :]
    gram = gs_ref[0:64, :]
    m2 = lax.dot_general(w2, cs.reshape(64, 1), (((1,), (0,)), ((), ())),
                         preferred_element_type=jnp.float32)[:, 0] / _R
    t = lax.dot_general(w2, gram, (((1,), (0,)), ((), ())),
                        preferred_element_type=jnp.float32)
    e2 = jnp.sum(t * w2, axis=1) / _R
    var2 = e2 - m2 * m2
    sc2 = g2_ref[0, :] / jnp.sqrt(var2 + _EPS)
    sh2 = b2_ref[0, :] - m2 * sc2

    y2 = lax.dot_general(x1, w2, (((1,), (1,)), ((), ())),
                         preferred_element_type=jnp.float32)
    x2 = jnp.maximum(y2 * sc2[None, :] + sh2[None, :], 0.0)
    y3 = lax.dot_general(x2, w3_ref[...], (((1,), (1,)), ((), ())),
                         preferred_element_type=jnp.float32)
    s3 = jnp.sum(y3, axis=0)
    q3 = jnp.sum(y3 * y3, axis=0)
    st = jnp.stack([s3, q3])
    m3_ref[...] = jnp.max(y3.reshape(_TR // K, K, 128), axis=1)

    @pl.when(pl.program_id(0) == 0)
    def _():
        st3_ref[...] = st

    @pl.when(pl.program_id(0) != 0)
    def _():
        st3_ref[...] += st


@jax.jit
def _mlp(G, Q, st1, gs, g1, b1, g2, b2, W2, W3):
    return pl.pallas_call(
        _mlp_body,
        grid=(_NT,),
        in_specs=[
            pl.BlockSpec((_TR, 128), lambda i: (i, 0)),
            pl.BlockSpec((_TR // K, 64), lambda i: (i, 0)),
            pl.BlockSpec((2, 64), lambda i: (0, 0)),
            pl.BlockSpec((65, 64), lambda i: (0, 0)),
            pl.BlockSpec((1, 64), lambda i: (0, 0)),
            pl.BlockSpec((1, 64), lambda i: (0, 0)),
            pl.BlockSpec((1, 64), lambda i: (0, 0)),
            pl.BlockSpec((1, 64), lambda i: (0, 0)),
            pl.BlockSpec((64, 64), lambda i: (0, 0)),
            pl.BlockSpec((128, 64), lambda i: (0, 0)),
        ],
        out_specs=[
            pl.BlockSpec((_TR // K, 128), lambda i: (i, 0)),
            pl.BlockSpec((2, 128), lambda i: (0, 0)),
        ],
        out_shape=[
            jax.ShapeDtypeStruct((B * S, 128), jnp.float32),
            jax.ShapeDtypeStruct((2, 128), jnp.float32),
        ],
    )(G, Q, st1, gs, g1, b1, g2, b2, W2, W3)


def _bn3_body(m3_ref, st3_ref, g3_ref, b3_ref, out_ref):
    sc3, sh3 = _bn_coeffs(st3_ref, g3_ref, b3_ref)
    o = jnp.maximum(m3_ref[...] * sc3[None, :] + sh3[None, :], 0.0)
    out_ref[0] = o.T


@jax.jit
def _bn3(m3, st3, g3, b3):
    return pl.pallas_call(
        _bn3_body,
        grid=(B,),
        in_specs=[
            pl.BlockSpec((S, 128), lambda i: (i, 0)),
            pl.BlockSpec((2, 128), lambda i: (0, 0)),
            pl.BlockSpec((1, 128), lambda i: (0, 0)),
            pl.BlockSpec((1, 128), lambda i: (0, 0)),
        ],
        out_specs=pl.BlockSpec((1, 128, S), lambda i: (i, 0, 0)),
        out_shape=jax.ShapeDtypeStruct((B, 128, S), jnp.float32),
    )(m3, st3, g3, b3)


def kernel(xyz, features, W1, g1, b1, W2, g2, b2, W3, g3, b3):
    xt = jnp.transpose(xyz, (0, 2, 1))
    fps_idx, cx, cy, cz = _fps(xt[:, 0], xt[:, 1], xt[:, 2])
    new_xyz = jnp.stack([cx, cy, cz], axis=-1)

    gidx = _ball_query_sc(
        xt[:, 0].reshape(-1), xt[:, 1].reshape(-1), xt[:, 2].reshape(-1),
        cx.reshape(-1), cy.reshape(-1), cz.reshape(-1))

    wx = W1[:, :3]
    wf = W1[:, 3:]
    T1 = _t1_table(xt, features, wx, wf).reshape(B * N, 128)
    Q = _q_table(jnp.stack([cx, cy, cz], axis=1), wx).reshape(B * S, 64)
    G = _gather_sc(T1, gidx)

    st1 = _stats1(G, Q)
    gs = _stats2(G, Q, st1, g1[None, :], b1[None, :])
    m3, st3 = _mlp(G, Q, st1, gs, g1[None, :], b1[None, :],
                   g2[None, :], b2[None, :], W2, W3)
    new_features = _bn3(m3, st3, g3[None, :], b3[None, :])
    return (new_xyz, new_features)


# speedup vs baseline: 2.7538x; 1.0167x over previous
import functools

import jax
import jax.numpy as jnp
from jax import lax
from jax.experimental import pallas as pl
from jax.experimental.pallas import tpu as pltpu
from jax.experimental.pallas import tpu_sc as plsc

B = 8
N = 4096
S = 1024
K = 32
RADIUS = 0.2


def _fps_body(x_ref, y_ref, z_ref, idx_ref, cx_ref, cy_ref, cz_ref, dist_ref):
    dist_ref[...] = jnp.full((B, N), 1e10, jnp.float32)
    lane = jax.lax.broadcasted_iota(jnp.int32, (B, 128), 1)
    NQ = 4
    QL = N // NQ

    def outer(j, far):
        z32 = jnp.zeros((B, 128), jnp.int32)
        zf = jnp.zeros((B, 128), jnp.float32)

        def inner(t, st):
            far, sidx, scx, scy, scz = st
            cx = jnp.zeros((B, 1), jnp.float32)
            cy = jnp.zeros((B, 1), jnp.float32)
            cz = jnp.zeros((B, 1), jnp.float32)
            for q in range(NQ):
                sl = pl.ds(q * QL, QL)
                io = jax.lax.broadcasted_iota(jnp.int32, (B, QL), 1) + q * QL
                m = io == far
                cx = cx + jnp.sum(jnp.where(m, x_ref[:, sl], 0.0), axis=1,
                                  keepdims=True)
                cy = cy + jnp.sum(jnp.where(m, y_ref[:, sl], 0.0), axis=1,
                                  keepdims=True)
                cz = cz + jnp.sum(jnp.where(m, z_ref[:, sl], 0.0), axis=1,
                                  keepdims=True)
            mx = jnp.full((B, 1), -1.0, jnp.float32)
            mi = jnp.full((B, 1), N, jnp.int32)
            for q in range(NQ):
                sl = pl.ds(q * QL, QL)
                io = jax.lax.broadcasted_iota(jnp.int32, (B, QL), 1) + q * QL
                dx = x_ref[:, sl] - cx
                dy = y_ref[:, sl] - cy
                dz = z_ref[:, sl] - cz
                d = dx * dx + dy * dy + dz * dz
                dq = jnp.minimum(dist_ref[:, sl], d)
                dist_ref[:, sl] = dq
                qmax = jnp.max(dq, axis=1, keepdims=True)
                qidx = jnp.min(jnp.where(dq == qmax, io, N), axis=1,
                               keepdims=True)
                take = qmax > mx
                mi = jnp.where(take, qidx, mi)
                mx = jnp.where(take, qmax, mx)
            oh = lane == t
            sidx = jnp.where(oh, far, sidx)
            scx = jnp.where(oh, cx, scx)
            scy = jnp.where(oh, cy, scy)
            scz = jnp.where(oh, cz, scz)
            return (mi, sidx, scx, scy, scz)

        far, sidx, scx, scy, scz = jax.lax.fori_loop(
            0, 128, inner, (far, z32, zf, zf, zf), unroll=False)
        col = pl.multiple_of(j * 128, 128)
        idx_ref[:, pl.ds(col, 128)] = sidx
        cx_ref[:, pl.ds(col, 128)] = scx
        cy_ref[:, pl.ds(col, 128)] = scy
        cz_ref[:, pl.ds(col, 128)] = scz
        return far

    jax.lax.fori_loop(0, S // 128, outer, jnp.zeros((B, 1), jnp.int32),
                      unroll=False)


@functools.partial(jax.jit, static_argnames=("interpret",))
def _fps(x, y, z, interpret=False):
    out_shapes = (
        jax.ShapeDtypeStruct((B, S), jnp.int32),
        jax.ShapeDtypeStruct((B, S), jnp.float32),
        jax.ShapeDtypeStruct((B, S), jnp.float32),
        jax.ShapeDtypeStruct((B, S), jnp.float32),
    )
    return pl.pallas_call(
        _fps_body,
        out_shape=out_shapes,
        scratch_shapes=[pltpu.VMEM((B, N), jnp.float32)],
        interpret=interpret,
    )(x, y, z)


_NW = 32
_RPW = (B * S) // _NW
_RBLK = 16
_CB = 16
_NBLK2 = N // (16 * _CB)


def _bq_body(px_h, py_h, pz_h, cx_h, cy_h, cz_h, out_h,
             x_v, y_v, z_v, cx_v, cy_v, cz_v, blk_v):
    w = lax.axis_index("s") * 2 + lax.axis_index("c")
    b = w // (_NW // B)
    row0 = w * _RPW
    pltpu.sync_copy(px_h.at[pl.ds(b * N, N)], x_v)
    pltpu.sync_copy(py_h.at[pl.ds(b * N, N)], y_v)
    pltpu.sync_copy(pz_h.at[pl.ds(b * N, N)], z_v)
    pltpu.sync_copy(cx_h.at[pl.ds(row0, _RPW)], cx_v)
    pltpu.sync_copy(cy_h.at[pl.ds(row0, _RPW)], cy_v)
    pltpu.sync_copy(cz_h.at[pl.ds(row0, _RPW)], cz_v)
    r2 = RADIUS * RADIUS
    iota16 = lax.iota(jnp.int32, 16)
    boff = b * N

    def do_block(blk, _):
        def do_row(r, _):
            sw = blk * _RBLK + r
            rowoff = r * K
            sv = jnp.full((16,), sw, jnp.int32)
            cxs = plsc.load_gather(cx_v, [sv])
            cys = plsc.load_gather(cy_v, [sv])
            czs = plsc.load_gather(cz_v, [sv])

            def cond(st):
                cnt, blk2 = st
                return (cnt < K) & (blk2 < _NBLK2)

            def step(st):
                cnt, blk2 = st
                base0 = blk2 * (16 * _CB)
                masks = []
                for c in range(_CB):
                    off = base0 + c * 16
                    dx = x_v[pl.ds(off, 16)] - cxs
                    dy = y_v[pl.ds(off, 16)] - cys
                    dz = z_v[pl.ds(off, 16)] - czs
                    masks.append((dx * dx + dy * dy + dz * dz) <= r2)
                basev = jnp.full((16,), cnt, jnp.int32)
                bases = []
                for c in range(_CB):
                    bases.append(basev)
                    basev = basev + plsc.all_reduce_population_count(masks[c])
                for c in range(_CB):
                    m = masks[c]
                    pc = plsc.cumsum(jnp.where(m, 1, 0).astype(jnp.int32))
                    pos = pc + (bases[c] - 1)
                    wmask = m & (pos < K)
                    vals = (boff + base0 + c * 16) + iota16
                    plsc.store_scatter(blk_v, [pos + rowoff], vals, mask=wmask)
                return (jnp.max(basev), blk2 + 1)

            cnt, _ = lax.while_loop(cond, step, (jnp.int32(0), jnp.int32(0)))

            first = plsc.load_gather(blk_v, [jnp.full((16,), rowoff, jnp.int32)])
            for h in range(2):
                off = rowoff + h * 16
                cur = blk_v[pl.ds(off, 16)]
                lanes = iota16 + (h * 16)
                blk_v[pl.ds(off, 16)] = jnp.where(lanes >= cnt, first, cur)
            return _

        lax.fori_loop(0, _RBLK, do_row, 0, unroll=False)
        pltpu.sync_copy(blk_v, out_h.at[pl.ds((row0 + blk * _RBLK) * K,
                                              _RBLK * K)])
        return _

    lax.fori_loop(0, _RPW // _RBLK, do_block, 0, unroll=False)


@jax.jit
def _ball_query_sc(px, py, pz, cx, cy, cz):
    mesh = plsc.VectorSubcoreMesh(core_axis_name="c", subcore_axis_name="s")
    f = pl.kernel(
        _bq_body,
        mesh=mesh,
        compiler_params=pltpu.CompilerParams(needs_layout_passes=False),
        out_type=jax.ShapeDtypeStruct((B * S * K,), jnp.int32),
        scratch_types=[
            pltpu.VMEM((N,), jnp.float32),
            pltpu.VMEM((N,), jnp.float32),
            pltpu.VMEM((N,), jnp.float32),
            pltpu.VMEM((_RPW,), jnp.float32),
            pltpu.VMEM((_RPW,), jnp.float32),
            pltpu.VMEM((_RPW,), jnp.float32),
            pltpu.VMEM((_RBLK * K,), jnp.int32),
        ],
    )
    return f(px, py, pz, cx, cy, cz)


_R = B * S * K
_EPS = 1e-5


def _t1_body(xt_ref, f_ref, wx_ref, wf_ref, out_ref):
    yx = lax.dot_general(xt_ref[0], wx_ref[...], (((0,), (1,)), ((), ())),
                         preferred_element_type=jnp.float32)
    yf = lax.dot_general(f_ref[0], wf_ref[...], (((0,), (1,)), ((), ())),
                         preferred_element_type=jnp.float32)
    out_ref[0] = jnp.concatenate(
        [yx + yf, jnp.zeros((512, 64), jnp.float32)], axis=1)


@jax.jit
def _t1_table(xt, features, wx, wf):
    return pl.pallas_call(
        _t1_body,
        grid=(B, N // 512),
        in_specs=[
            pl.BlockSpec((1, 3, 512), lambda b, j: (b, 0, j)),
            pl.BlockSpec((1, 64, 512), lambda b, j: (b, 0, j)),
            pl.BlockSpec((64, 3), lambda b, j: (0, 0)),
            pl.BlockSpec((64, 64), lambda b, j: (0, 0)),
        ],
        out_specs=pl.BlockSpec((1, 512, 128), lambda b, j: (b, j, 0)),
        out_shape=jax.ShapeDtypeStruct((B, N, 128), jnp.float32),
    )(xt, features, wx, wf)


def _q_body(c_ref, wx_ref, out_ref):
    out_ref[0] = lax.dot_general(c_ref[0], wx_ref[...], (((0,), (1,)), ((), ())),
                                 preferred_element_type=jnp.float32)


@jax.jit
def _q_table(cstack, wx):
    return pl.pallas_call(
        _q_body,
        grid=(B, S // 512),
        in_specs=[
            pl.BlockSpec((1, 3, 512), lambda b, j: (b, 0, j)),
            pl.BlockSpec((64, 3), lambda b, j: (0, 0)),
        ],
        out_specs=pl.BlockSpec((1, 512, 64), lambda b, j: (b, j, 0)),
        out_shape=jax.ShapeDtypeStruct((B, S, 64), jnp.float32),
    )(cstack, wx)


_GCH = _R // _NW
_GBLK = 128


def _gather_body(t1_h, gidx_h, out_h, idx_v, buf_v, semg, semw):
    w = lax.axis_index("s") * 2 + lax.axis_index("c")
    base = w * _GCH
    nch = _GCH // _GBLK
    pltpu.sync_copy(gidx_h.at[pl.ds(base, _GCH)], idx_v)

    pltpu.async_copy(t1_h.at[idx_v.at[pl.ds(0, _GBLK)]], buf_v.at[0],
                     semg).wait()

    def do_chunk(t, _):
        cur = t & 1
        pltpu.async_copy(buf_v.at[1 - cur],
                         out_h.at[pl.ds(base + (t - 1) * _GBLK, _GBLK)], semw)
        pltpu.async_copy(t1_h.at[idx_v.at[pl.ds(t * _GBLK, _GBLK)]],
                         buf_v.at[cur], semg).wait()
        pltpu.make_async_copy(
            buf_v.at[cur], out_h.at[pl.ds(base, _GBLK)], semw).wait()
        return _

    lax.fori_loop(1, nch, do_chunk, 0, unroll=False)
    pltpu.sync_copy(buf_v.at[(nch - 1) & 1],
                    out_h.at[pl.ds(base + (nch - 1) * _GBLK, _GBLK)])


@jax.jit
def _gather_sc(t1_flat, gidx):
    mesh = plsc.VectorSubcoreMesh(core_axis_name="c", subcore_axis_name="s")
    f = pl.kernel(
        _gather_body,
        mesh=mesh,
        compiler_params=pltpu.CompilerParams(needs_layout_passes=False),
        out_type=jax.ShapeDtypeStruct((_R, 128), jnp.float32),
        scratch_types=[
            pltpu.VMEM((_GCH,), jnp.int32),
            pltpu.VMEM((2, _GBLK, 128), jnp.float32),
            pltpu.SemaphoreType.DMA,
            pltpu.SemaphoreType.DMA,
        ],
    )
    return f(t1_flat, gidx)


_TR = 2048
_NT = _R // _TR


def _bn_coeffs(st_ref, g_ref, b_ref):
    s = st_ref[0, :]
    sq = st_ref[1, :]
    m = s / _R
    var = sq / _R - m * m
    inv = g_ref[0, :] / jnp.sqrt(var + _EPS)
    return inv, b_ref[0, :] - m * inv


def _x1_tile(g_ref, q_ref, st1_ref, g1_ref, b1_ref):
    sc1, sh1 = _bn_coeffs(st1_ref, g1_ref, b1_ref)
    y1 = g_ref[:, 0:64].reshape(_TR // K, K, 64) - q_ref[...][:, None, :]
    x1 = jnp.maximum(y1 * sc1[None, None, :] + sh1[None, None, :], 0.0)
    return x1.reshape(_TR, 64)


def _stats1_body(g_ref, q_ref, out_ref):
    y = g_ref[:, 0:64].reshape(_TR // K, K, 64) - q_ref[...][:, None, :]
    s = jnp.sum(y, axis=(0, 1))
    sq = jnp.sum(y * y, axis=(0, 1))
    st = jnp.stack([s, sq])

    @pl.when(pl.program_id(0) == 0)
    def _():
        out_ref[...] = st

    @pl.when(pl.program_id(0) != 0)
    def _():
        out_ref[...] += st


@jax.jit
def _stats1(G, Q):
    return pl.pallas_call(
        _stats1_body,
        grid=(_NT,),
        in_specs=[
            pl.BlockSpec((_TR, 128), lambda i: (i, 0)),
            pl.BlockSpec((_TR // K, 64), lambda i: (i, 0)),
        ],
        out_specs=pl.BlockSpec((2, 64), lambda i: (0, 0)),
        out_shape=jax.ShapeDtypeStruct((2, 64), jnp.float32),
    )(G, Q)


def _stats2_body(g_ref, q_ref, st1_ref, g1_ref, b1_ref, out_ref):
    x1 = _x1_tile(g_ref, q_ref, st1_ref, g1_ref, b1_ref)
    gram = lax.dot_general(x1, x1, (((0,), (0,)), ((), ())),
                           preferred_element_type=jnp.float32)
    cs = jnp.sum(x1, axis=0)
    st = jnp.concatenate([gram, cs[None, :]], axis=0)

    @pl.when(pl.program_id(0) == 0)
    def _():
        out_ref[...] = st

    @pl.when(pl.program_id(0) != 0)
    def _():
        out_ref[...] += st


@jax.jit
def _stats2(G, Q, st1, g1, b1):
    return pl.pallas_call(
        _stats2_body,
        grid=(_NT,),
        in_specs=[
            pl.BlockSpec((_TR, 128), lambda i: (i, 0)),
            pl.BlockSpec((_TR // K, 64), lambda i: (i, 0)),
            pl.BlockSpec((2, 64), lambda i: (0, 0)),
            pl.BlockSpec((1, 64), lambda i: (0, 0)),
            pl.BlockSpec((1, 64), lambda i: (0, 0)),
        ],
        out_specs=pl.BlockSpec((65, 64), lambda i: (0, 0)),
        out_shape=jax.ShapeDtypeStruct((65, 64), jnp.float32),
    )(G, Q, st1, g1, b1)


def _mlp_body(g_ref, q_ref, st1_ref, gs_ref, g1_ref, b1_ref, g2_ref, b2_ref,
              w2_ref, w3_ref, m3_ref, st3_ref):
    x1 = _x1_tile(g_ref, q_ref, st1_ref, g1_ref, b1_ref)

    w2 = w2_ref[...]
    cs = gs_ref[64, :]
    gram = gs_ref[0:64, :]
    m2 = lax.dot_general(w2, cs.reshape(64, 1), (((1,), (0,)), ((), ())),
                         preferred_element_type=jnp.float32)[:, 0] / _R
    t = lax.dot_general(w2, gram, (((1,), (0,)), ((), ())),
                        preferred_element_type=jnp.float32)
    e2 = jnp.sum(t * w2, axis=1) / _R
    var2 = e2 - m2 * m2
    sc2 = g2_ref[0, :] / jnp.sqrt(var2 + _EPS)
    sh2 = b2_ref[0, :] - m2 * sc2

    y2 = lax.dot_general(x1, w2, (((1,), (1,)), ((), ())),
                         preferred_element_type=jnp.float32)
    x2 = jnp.maximum(y2 * sc2[None, :] + sh2[None, :], 0.0)
    y3 = lax.dot_general(x2, w3_ref[...], (((1,), (1,)), ((), ())),
                         preferred_element_type=jnp.float32)
    s3 = jnp.sum(y3, axis=0)
    q3 = jnp.sum(y3 * y3, axis=0)
    st = jnp.stack([s3, q3])
    m3_ref[...] = jnp.max(y3.reshape(_TR // K, K, 128), axis=1)

    @pl.when(pl.program_id(0) == 0)
    def _():
        st3_ref[...] = st

    @pl.when(pl.program_id(0) != 0)
    def _():
        st3_ref[...] += st


@jax.jit
def _mlp(G, Q, st1, gs, g1, b1, g2, b2, W2, W3):
    return pl.pallas_call(
        _mlp_body,
        grid=(_NT,),
        in_specs=[
            pl.BlockSpec((_TR, 128), lambda i: (i, 0)),
            pl.BlockSpec((_TR // K, 64), lambda i: (i, 0)),
            pl.BlockSpec((2, 64), lambda i: (0, 0)),
            pl.BlockSpec((65, 64), lambda i: (0, 0)),
            pl.BlockSpec((1, 64), lambda i: (0, 0)),
            pl.BlockSpec((1, 64), lambda i: (0, 0)),
            pl.BlockSpec((1, 64), lambda i: (0, 0)),
            pl.BlockSpec((1, 64), lambda i: (0, 0)),
            pl.BlockSpec((64, 64), lambda i: (0, 0)),
            pl.BlockSpec((128, 64), lambda i: (0, 0)),
        ],
        out_specs=[
            pl.BlockSpec((_TR // K, 128), lambda i: (i, 0)),
            pl.BlockSpec((2, 128), lambda i: (0, 0)),
        ],
        out_shape=[
            jax.ShapeDtypeStruct((B * S, 128), jnp.float32),
            jax.ShapeDtypeStruct((2, 128), jnp.float32),
        ],
    )(G, Q, st1, gs, g1, b1, g2, b2, W2, W3)


def _bn3_body(m3_ref, st3_ref, g3_ref, b3_ref, out_ref):
    sc3, sh3 = _bn_coeffs(st3_ref, g3_ref, b3_ref)
    o = jnp.maximum(m3_ref[...] * sc3[None, :] + sh3[None, :], 0.0)
    out_ref[0] = o.T


@jax.jit
def _bn3(m3, st3, g3, b3):
    return pl.pallas_call(
        _bn3_body,
        grid=(B,),
        in_specs=[
            pl.BlockSpec((S, 128), lambda i: (i, 0)),
            pl.BlockSpec((2, 128), lambda i: (0, 0)),
            pl.BlockSpec((1, 128), lambda i: (0, 0)),
            pl.BlockSpec((1, 128), lambda i: (0, 0)),
        ],
        out_specs=pl.BlockSpec((1, 128, S), lambda i: (i, 0, 0)),
        out_shape=jax.ShapeDtypeStruct((B, 128, S), jnp.float32),
    )(m3, st3, g3, b3)


def kernel(xyz, features, W1, g1, b1, W2, g2, b2, W3, g3, b3):
    xt = jnp.transpose(xyz, (0, 2, 1))
    fps_idx, cx, cy, cz = _fps(xt[:, 0], xt[:, 1], xt[:, 2])
    new_xyz = jnp.stack([cx, cy, cz], axis=-1)

    gidx = _ball_query_sc(
        xt[:, 0].reshape(-1), xt[:, 1].reshape(-1), xt[:, 2].reshape(-1),
        cx.reshape(-1), cy.reshape(-1), cz.reshape(-1))

    wx = W1[:, :3]
    wf = W1[:, 3:]
    T1 = _t1_table(xt, features, wx, wf).reshape(B * N, 128)
    Q = _q_table(jnp.stack([cx, cy, cz], axis=1), wx).reshape(B * S, 64)
    G = _gather_sc(T1, gidx)

    st1 = _stats1(G, Q)
    gs = _stats2(G, Q, st1, g1[None, :], b1[None, :])
    m3, st3 = _mlp(G, Q, st1, gs, g1[None, :], b1[None, :],
                   g2[None, :], b2[None, :], W2, W3)
    new_features = _bn3(m3, st3, g3[None, :], b3[None, :])
    return (new_xyz, new_features)


# speedup vs baseline: 2.7691x; 1.0055x over previous
import jax
import jax.numpy as jnp
from jax import lax
from jax.experimental import pallas as pl
from jax.experimental.pallas import tpu as pltpu
from jax.experimental.pallas import tpu_sc as plsc

B = 8
N = 4096
S = 1024
K = 32
RADIUS = 0.2


def _fps_body(x_ref, y_ref, z_ref, idx_ref, cx_ref, cy_ref, cz_ref, dist_ref):
    dist_ref[...] = jnp.full((B, N), 1e10, jnp.float32)
    lane = jax.lax.broadcasted_iota(jnp.int32, (B, 128), 1)
    NQ = 4
    QL = N // NQ

    def outer(j, far):
        z32 = jnp.zeros((B, 128), jnp.int32)
        zf = jnp.zeros((B, 128), jnp.float32)

        def inner(t, st):
            far, sidx, scx, scy, scz = st
            cx = jnp.zeros((B, 1), jnp.float32)
            cy = jnp.zeros((B, 1), jnp.float32)
            cz = jnp.zeros((B, 1), jnp.float32)
            for q in range(NQ):
                sl = pl.ds(q * QL, QL)
                io = jax.lax.broadcasted_iota(jnp.int32, (B, QL), 1) + q * QL
                m = io == far
                cx = cx + jnp.sum(jnp.where(m, x_ref[:, sl], 0.0), axis=1,
                                  keepdims=True)
                cy = cy + jnp.sum(jnp.where(m, y_ref[:, sl], 0.0), axis=1,
                                  keepdims=True)
                cz = cz + jnp.sum(jnp.where(m, z_ref[:, sl], 0.0), axis=1,
                                  keepdims=True)
            mx = jnp.full((B, 1), -1.0, jnp.float32)
            mi = jnp.full((B, 1), N, jnp.int32)
            for q in range(NQ):
                sl = pl.ds(q * QL, QL)
                io = jax.lax.broadcasted_iota(jnp.int32, (B, QL), 1) + q * QL
                dx = x_ref[:, sl] - cx
                dy = y_ref[:, sl] - cy
                dz = z_ref[:, sl] - cz
                d = dx * dx + dy * dy + dz * dz
                dq = jnp.minimum(dist_ref[:, sl], d)
                dist_ref[:, sl] = dq
                qmax = jnp.max(dq, axis=1, keepdims=True)
                qidx = jnp.min(jnp.where(dq == qmax, io, N), axis=1,
                               keepdims=True)
                take = qmax > mx
                mi = jnp.where(take, qidx, mi)
                mx = jnp.where(take, qmax, mx)
            oh = lane == t
            sidx = jnp.where(oh, far, sidx)
            scx = jnp.where(oh, cx, scx)
            scy = jnp.where(oh, cy, scy)
            scz = jnp.where(oh, cz, scz)
            return (mi, sidx, scx, scy, scz)

        far, sidx, scx, scy, scz = jax.lax.fori_loop(
            0, 128, inner, (far, z32, zf, zf, zf), unroll=False)
        col = pl.multiple_of(j * 128, 128)
        idx_ref[:, pl.ds(col, 128)] = sidx
        cx_ref[:, pl.ds(col, 128)] = scx
        cy_ref[:, pl.ds(col, 128)] = scy
        cz_ref[:, pl.ds(col, 128)] = scz
        return far

    jax.lax.fori_loop(0, S // 128, outer, jnp.zeros((B, 1), jnp.int32),
                      unroll=False)


@jax.jit
def _fps(x, y, z):
    out_shapes = (
        jax.ShapeDtypeStruct((B, S), jnp.int32),
        jax.ShapeDtypeStruct((B, S), jnp.float32),
        jax.ShapeDtypeStruct((B, S), jnp.float32),
        jax.ShapeDtypeStruct((B, S), jnp.float32),
    )
    return pl.pallas_call(
        _fps_body,
        out_shape=out_shapes,
        scratch_shapes=[pltpu.VMEM((B, N), jnp.float32)],
    )(x, y, z)


_NW = 32
_RPW = (B * S) // _NW
_RBLK = 16
_CB = 16
_NBLK2 = N // (16 * _CB)


def _bq_body(px_h, py_h, pz_h, cx_h, cy_h, cz_h, out_h,
             x_v, y_v, z_v, cx_v, cy_v, cz_v, blk_v):
    w = lax.axis_index("s") * 2 + lax.axis_index("c")
    b = w // (_NW // B)
    row0 = w * _RPW
    pltpu.sync_copy(px_h.at[pl.ds(b * N, N)], x_v)
    pltpu.sync_copy(py_h.at[pl.ds(b * N, N)], y_v)
    pltpu.sync_copy(pz_h.at[pl.ds(b * N, N)], z_v)
    pltpu.sync_copy(cx_h.at[pl.ds(row0, _RPW)], cx_v)
    pltpu.sync_copy(cy_h.at[pl.ds(row0, _RPW)], cy_v)
    pltpu.sync_copy(cz_h.at[pl.ds(row0, _RPW)], cz_v)
    r2 = RADIUS * RADIUS
    iota16 = lax.iota(jnp.int32, 16)
    boff = b * N

    def do_block(blk, _):
        def do_row(r, _):
            sw = blk * _RBLK + r
            rowoff = r * K
            sv = jnp.full((16,), sw, jnp.int32)
            cxs = plsc.load_gather(cx_v, [sv])
            cys = plsc.load_gather(cy_v, [sv])
            czs = plsc.load_gather(cz_v, [sv])

            def cond(st):
                cnt, blk2 = st
                return (cnt < K) & (blk2 < _NBLK2)

            def step(st):
                cnt, blk2 = st
                base0 = blk2 * (16 * _CB)
                masks = []
                for c in range(_CB):
                    off = base0 + c * 16
                    dx = x_v[pl.ds(off, 16)] - cxs
                    dy = y_v[pl.ds(off, 16)] - cys
                    dz = z_v[pl.ds(off, 16)] - czs
                    masks.append((dx * dx + dy * dy + dz * dz) <= r2)
                basev = jnp.full((16,), cnt, jnp.int32)
                bases = []
                for c in range(_CB):
                    bases.append(basev)
                    basev = basev + plsc.all_reduce_population_count(masks[c])
                for c in range(_CB):
                    m = masks[c]
                    pc = plsc.cumsum(jnp.where(m, 1, 0).astype(jnp.int32))
                    pos = pc + (bases[c] - 1)
                    wmask = m & (pos < K)
                    vals = (boff + base0 + c * 16) + iota16
                    plsc.store_scatter(blk_v, [pos + rowoff], vals, mask=wmask)
                return (jnp.max(basev), blk2 + 1)

            cnt, _ = lax.while_loop(cond, step, (jnp.int32(0), jnp.int32(0)))

            first = plsc.load_gather(blk_v, [jnp.full((16,), rowoff, jnp.int32)])
            for h in range(2):
                off = rowoff + h * 16
                cur = blk_v[pl.ds(off, 16)]
                lanes = iota16 + (h * 16)
                blk_v[pl.ds(off, 16)] = jnp.where(lanes >= cnt, first, cur)
            return _

        lax.fori_loop(0, _RBLK, do_row, 0, unroll=False)
        pltpu.sync_copy(blk_v, out_h.at[pl.ds((row0 + blk * _RBLK) * K,
                                              _RBLK * K)])
        return _

    lax.fori_loop(0, _RPW // _RBLK, do_block, 0, unroll=False)


@jax.jit
def _ball_query_sc(px, py, pz, cx, cy, cz):
    mesh = plsc.VectorSubcoreMesh(core_axis_name="c", subcore_axis_name="s")
    f = pl.kernel(
        _bq_body,
        mesh=mesh,
        compiler_params=pltpu.CompilerParams(needs_layout_passes=False),
        out_type=jax.ShapeDtypeStruct((B * S * K,), jnp.int32),
        scratch_types=[
            pltpu.VMEM((N,), jnp.float32),
            pltpu.VMEM((N,), jnp.float32),
            pltpu.VMEM((N,), jnp.float32),
            pltpu.VMEM((_RPW,), jnp.float32),
            pltpu.VMEM((_RPW,), jnp.float32),
            pltpu.VMEM((_RPW,), jnp.float32),
            pltpu.VMEM((_RBLK * K,), jnp.int32),
        ],
    )
    return f(px, py, pz, cx, cy, cz)


_R = B * S * K
_EPS = 1e-5


def _t1_body(xt_ref, f_ref, wx_ref, wf_ref, out_ref):
    yx = lax.dot_general(xt_ref[0], wx_ref[...], (((0,), (1,)), ((), ())),
                         preferred_element_type=jnp.float32)
    yf = lax.dot_general(f_ref[0], wf_ref[...], (((0,), (1,)), ((), ())),
                         preferred_element_type=jnp.float32)
    out_ref[0] = jnp.concatenate(
        [yx + yf, jnp.zeros((512, 64), jnp.float32)], axis=1)


@jax.jit
def _t1_table(xt, features, wx, wf):
    return pl.pallas_call(
        _t1_body,
        grid=(B, N // 512),
        in_specs=[
            pl.BlockSpec((1, 3, 512), lambda b, j: (b, 0, j)),
            pl.BlockSpec((1, 64, 512), lambda b, j: (b, 0, j)),
            pl.BlockSpec((64, 3), lambda b, j: (0, 0)),
            pl.BlockSpec((64, 64), lambda b, j: (0, 0)),
        ],
        out_specs=pl.BlockSpec((1, 512, 128), lambda b, j: (b, j, 0)),
        out_shape=jax.ShapeDtypeStruct((B, N, 128), jnp.float32),
    )(xt, features, wx, wf)


def _q_body(c_ref, wx_ref, out_ref):
    out_ref[0] = lax.dot_general(c_ref[0], wx_ref[...], (((0,), (1,)), ((), ())),
                                 preferred_element_type=jnp.float32)


@jax.jit
def _q_table(cstack, wx):
    return pl.pallas_call(
        _q_body,
        grid=(B, S // 512),
        in_specs=[
            pl.BlockSpec((1, 3, 512), lambda b, j: (b, 0, j)),
            pl.BlockSpec((64, 3), lambda b, j: (0, 0)),
        ],
        out_specs=pl.BlockSpec((1, 512, 64), lambda b, j: (b, j, 0)),
        out_shape=jax.ShapeDtypeStruct((B, S, 64), jnp.float32),
    )(cstack, wx)


_GCH = _R // _NW
_GBLK = 128


def _gather_body(t1_h, gidx_h, out_h, idx_v, buf_v, semg, semw):
    w = lax.axis_index("s") * 2 + lax.axis_index("c")
    base = w * _GCH
    nch = _GCH // _GBLK
    pltpu.sync_copy(gidx_h.at[pl.ds(base, _GCH)], idx_v)

    pltpu.async_copy(t1_h.at[idx_v.at[pl.ds(0, _GBLK)]], buf_v.at[0],
                     semg).wait()

    def do_chunk(t, _):
        cur = t & 1
        pltpu.async_copy(buf_v.at[1 - cur],
                         out_h.at[pl.ds(base + (t - 1) * _GBLK, _GBLK)], semw)
        pltpu.async_copy(t1_h.at[idx_v.at[pl.ds(t * _GBLK, _GBLK)]],
                         buf_v.at[cur], semg).wait()
        pltpu.make_async_copy(
            buf_v.at[cur], out_h.at[pl.ds(base, _GBLK)], semw).wait()
        return _

    lax.fori_loop(1, nch, do_chunk, 0, unroll=False)
    pltpu.sync_copy(buf_v.at[(nch - 1) & 1],
                    out_h.at[pl.ds(base + (nch - 1) * _GBLK, _GBLK)])


@jax.jit
def _gather_sc(t1_flat, gidx):
    mesh = plsc.VectorSubcoreMesh(core_axis_name="c", subcore_axis_name="s")
    f = pl.kernel(
        _gather_body,
        mesh=mesh,
        compiler_params=pltpu.CompilerParams(needs_layout_passes=False),
        out_type=jax.ShapeDtypeStruct((_R, 128), jnp.float32),
        scratch_types=[
            pltpu.VMEM((_GCH,), jnp.int32),
            pltpu.VMEM((2, _GBLK, 128), jnp.float32),
            pltpu.SemaphoreType.DMA,
            pltpu.SemaphoreType.DMA,
        ],
    )
    return f(t1_flat, gidx)


_TR = 2048
_NT = _R // _TR


def _bn_coeffs(st_ref, g_ref, b_ref):
    s = st_ref[0, :]
    sq = st_ref[1, :]
    m = s / _R
    var = sq / _R - m * m
    inv = g_ref[0, :] / jnp.sqrt(var + _EPS)
    return inv, b_ref[0, :] - m * inv


def _x1_tile(g_ref, q_ref, st1_ref, g1_ref, b1_ref):
    sc1, sh1 = _bn_coeffs(st1_ref, g1_ref, b1_ref)
    y1 = g_ref[:, 0:64].reshape(_TR // K, K, 64) - q_ref[...][:, None, :]
    x1 = jnp.maximum(y1 * sc1[None, None, :] + sh1[None, None, :], 0.0)
    return x1.reshape(_TR, 64)


def _stats1_body(g_ref, q_ref, out_ref):
    y = g_ref[:, 0:64].reshape(_TR // K, K, 64) - q_ref[...][:, None, :]
    s = jnp.sum(y, axis=(0, 1))
    sq = jnp.sum(y * y, axis=(0, 1))
    st = jnp.stack([s, sq])

    @pl.when(pl.program_id(0) == 0)
    def _():
        out_ref[...] = st

    @pl.when(pl.program_id(0) != 0)
    def _():
        out_ref[...] += st


@jax.jit
def _stats1(G, Q):
    return pl.pallas_call(
        _stats1_body,
        grid=(_NT,),
        in_specs=[
            pl.BlockSpec((_TR, 128), lambda i: (i, 0)),
            pl.BlockSpec((_TR // K, 64), lambda i: (i, 0)),
        ],
        out_specs=pl.BlockSpec((2, 64), lambda i: (0, 0)),
        out_shape=jax.ShapeDtypeStruct((2, 64), jnp.float32),
    )(G, Q)


def _stats2_body(g_ref, q_ref, st1_ref, g1_ref, b1_ref, out_ref):
    x1 = _x1_tile(g_ref, q_ref, st1_ref, g1_ref, b1_ref)
    gram = lax.dot_general(x1, x1, (((0,), (0,)), ((), ())),
                           preferred_element_type=jnp.float32)
    cs = jnp.sum(x1, axis=0)
    st = jnp.concatenate([gram, cs[None, :]], axis=0)

    @pl.when(pl.program_id(0) == 0)
    def _():
        out_ref[...] = st

    @pl.when(pl.program_id(0) != 0)
    def _():
        out_ref[...] += st


@jax.jit
def _stats2(G, Q, st1, g1, b1):
    return pl.pallas_call(
        _stats2_body,
        grid=(_NT,),
        in_specs=[
            pl.BlockSpec((_TR, 128), lambda i: (i, 0)),
            pl.BlockSpec((_TR // K, 64), lambda i: (i, 0)),
            pl.BlockSpec((2, 64), lambda i: (0, 0)),
            pl.BlockSpec((1, 64), lambda i: (0, 0)),
            pl.BlockSpec((1, 64), lambda i: (0, 0)),
        ],
        out_specs=pl.BlockSpec((65, 64), lambda i: (0, 0)),
        out_shape=jax.ShapeDtypeStruct((65, 64), jnp.float32),
    )(G, Q, st1, g1, b1)


def _mlp_body(g_ref, q_ref, st1_ref, gs_ref, g1_ref, b1_ref, g2_ref, b2_ref,
              w2_ref, w3_ref, m3_ref, st3_ref):
    x1 = _x1_tile(g_ref, q_ref, st1_ref, g1_ref, b1_ref)

    w2 = w2_ref[...]
    cs = gs_ref[64, :]
    gram = gs_ref[0:64, :]
    m2 = lax.dot_general(w2, cs.reshape(64, 1), (((1,), (0,)), ((), ())),
                         preferred_element_type=jnp.float32)[:, 0] / _R
    t = lax.dot_general(w2, gram, (((1,), (0,)), ((), ())),
                        preferred_element_type=jnp.float32)
    e2 = jnp.sum(t * w2, axis=1) / _R
    var2 = e2 - m2 * m2
    sc2 = g2_ref[0, :] / jnp.sqrt(var2 + _EPS)
    sh2 = b2_ref[0, :] - m2 * sc2

    y2 = lax.dot_general(x1, w2, (((1,), (1,)), ((), ())),
                         preferred_element_type=jnp.float32)
    x2 = jnp.maximum(y2 * sc2[None, :] + sh2[None, :], 0.0)
    y3 = lax.dot_general(x2, w3_ref[...], (((1,), (1,)), ((), ())),
                         preferred_element_type=jnp.float32)
    s3 = jnp.sum(y3, axis=0)
    q3 = jnp.sum(y3 * y3, axis=0)
    st = jnp.stack([s3, q3])
    m3_ref[...] = jnp.max(y3.reshape(_TR // K, K, 128), axis=1)

    @pl.when(pl.program_id(0) == 0)
    def _():
        st3_ref[...] = st

    @pl.when(pl.program_id(0) != 0)
    def _():
        st3_ref[...] += st


@jax.jit
def _mlp(G, Q, st1, gs, g1, b1, g2, b2, W2, W3):
    return pl.pallas_call(
        _mlp_body,
        grid=(_NT,),
        in_specs=[
            pl.BlockSpec((_TR, 128), lambda i: (i, 0)),
            pl.BlockSpec((_TR // K, 64), lambda i: (i, 0)),
            pl.BlockSpec((2, 64), lambda i: (0, 0)),
            pl.BlockSpec((65, 64), lambda i: (0, 0)),
            pl.BlockSpec((1, 64), lambda i: (0, 0)),
            pl.BlockSpec((1, 64), lambda i: (0, 0)),
            pl.BlockSpec((1, 64), lambda i: (0, 0)),
            pl.BlockSpec((1, 64), lambda i: (0, 0)),
            pl.BlockSpec((64, 64), lambda i: (0, 0)),
            pl.BlockSpec((128, 64), lambda i: (0, 0)),
        ],
        out_specs=[
            pl.BlockSpec((_TR // K, 128), lambda i: (i, 0)),
            pl.BlockSpec((2, 128), lambda i: (0, 0)),
        ],
        out_shape=[
            jax.ShapeDtypeStruct((B * S, 128), jnp.float32),
            jax.ShapeDtypeStruct((2, 128), jnp.float32),
        ],
    )(G, Q, st1, gs, g1, b1, g2, b2, W2, W3)


def _bn3_body(m3_ref, st3_ref, g3_ref, b3_ref, out_ref):
    sc3, sh3 = _bn_coeffs(st3_ref, g3_ref, b3_ref)
    o = jnp.maximum(m3_ref[...] * sc3[None, :] + sh3[None, :], 0.0)
    out_ref[0] = o.T


@jax.jit
def _bn3(m3, st3, g3, b3):
    return pl.pallas_call(
        _bn3_body,
        grid=(B,),
        in_specs=[
            pl.BlockSpec((S, 128), lambda i: (i, 0)),
            pl.BlockSpec((2, 128), lambda i: (0, 0)),
            pl.BlockSpec((1, 128), lambda i: (0, 0)),
            pl.BlockSpec((1, 128), lambda i: (0, 0)),
        ],
        out_specs=pl.BlockSpec((1, 128, S), lambda i: (i, 0, 0)),
        out_shape=jax.ShapeDtypeStruct((B, 128, S), jnp.float32),
    )(m3, st3, g3, b3)


def kernel(xyz, features, W1, g1, b1, W2, g2, b2, W3, g3, b3):
    xt = jnp.transpose(xyz, (0, 2, 1))
    fps_idx, cx, cy, cz = _fps(xt[:, 0], xt[:, 1], xt[:, 2])
    new_xyz = jnp.stack([cx, cy, cz], axis=-1)

    gidx = _ball_query_sc(
        xt[:, 0].reshape(-1), xt[:, 1].reshape(-1), xt[:, 2].reshape(-1),
        cx.reshape(-1), cy.reshape(-1), cz.reshape(-1))

    wx = W1[:, :3]
    wf = W1[:, 3:]
    T1 = _t1_table(xt, features, wx, wf).reshape(B * N, 128)
    Q = _q_table(jnp.stack([cx, cy, cz], axis=1), wx).reshape(B * S, 64)
    G = _gather_sc(T1, gidx)

    st1 = _stats1(G, Q)
    gs = _stats2(G, Q, st1, g1[None, :], b1[None, :])
    m3, st3 = _mlp(G, Q, st1, gs, g1[None, :], b1[None, :],
                   g2[None, :], b2[None, :], W2, W3)
    new_features = _bn3(m3, st3, g3[None, :], b3[None, :])
    return (new_xyz, new_features)
